# baseline (speedup 1.0000x reference)
import functools

import jax
import jax.numpy as jnp
from jax.experimental import pallas as pl
from jax.experimental.pallas import tpu as pltpu


def _add_emb_kernel(x_ref, table_ref, o_ref, emb_ref, *, F, N, D):
    @pl.when(pl.program_id(0) == 0)
    def _compute_emb():
        t = table_ref[0:F, :]
        norm = jnp.sqrt(jnp.sum(t * t, axis=-1, keepdims=True))
        scale = jnp.where(norm > 1.0, 1.0 / jnp.maximum(norm, 1e-12), 1.0)
        t = t * scale
        r = jax.lax.broadcasted_iota(jnp.int32, (F, F), 0)
        c = jax.lax.broadcasted_iota(jnp.int32, (F, F), 1)
        perm = (c == (F - 1 - r)).astype(jnp.float32)
        emb = jnp.dot(perm, t, preferred_element_type=jnp.float32)
        emb_ref[...] = jnp.broadcast_to(emb[:, None, :], (F, N, D))

    o_ref[...] = x_ref[...] + emb_ref[...][None]


def kernel(x, in_F, out_F, table):
    B, F, N, D = x.shape
    BB = 8
    return pl.pallas_call(
        functools.partial(_add_emb_kernel, F=F, N=N, D=D),
        grid=(B // BB,),
        in_specs=[
            pl.BlockSpec((BB, F, N, D), lambda i: (i, 0, 0, 0)),
            pl.BlockSpec((table.shape[0], D), lambda i: (0, 0)),
        ],
        out_specs=pl.BlockSpec((BB, F, N, D), lambda i: (i, 0, 0, 0)),
        out_shape=jax.ShapeDtypeStruct(x.shape, x.dtype),
        scratch_shapes=[pltpu.VMEM((F, N, D), jnp.float32)],
    )(x, table)


# speedup vs baseline: 1.0115x; 1.0115x over previous
import functools

import jax
import jax.numpy as jnp
from jax.experimental import pallas as pl
from jax.experimental.pallas import tpu as pltpu


def _add_emb_kernel(x_ref, table_ref, o_ref, emb_ref, *, F, N, D):
    @pl.when(pl.program_id(0) == 0)
    def _compute_emb():
        t = table_ref[0:F, :]
        norm = jnp.sqrt(jnp.sum(t * t, axis=-1, keepdims=True))
        scale = jnp.where(norm > 1.0, 1.0 / jnp.maximum(norm, 1e-12), 1.0)
        t = t * scale
        r = jax.lax.broadcasted_iota(jnp.int32, (F, F), 0)
        c = jax.lax.broadcasted_iota(jnp.int32, (F, F), 1)
        perm = (c == (F - 1 - r)).astype(jnp.float32)
        emb = jnp.dot(perm, t, preferred_element_type=jnp.float32)
        emb_ref[...] = jnp.broadcast_to(emb[:, None, :], (F, N, D))

    o_ref[...] = x_ref[...] + emb_ref[...][None]


def kernel(x, in_F, out_F, table):
    B, F, N, D = x.shape
    BB = 16
    return pl.pallas_call(
        functools.partial(_add_emb_kernel, F=F, N=N, D=D),
        grid=(B // BB,),
        in_specs=[
            pl.BlockSpec((BB, F, N, D), lambda i: (i, 0, 0, 0)),
            pl.BlockSpec((table.shape[0], D), lambda i: (0, 0)),
        ],
        out_specs=pl.BlockSpec((BB, F, N, D), lambda i: (i, 0, 0, 0)),
        out_shape=jax.ShapeDtypeStruct(x.shape, x.dtype),
        scratch_shapes=[pltpu.VMEM((F, N, D), jnp.float32)],
    )(x, table)


# speedup vs baseline: 1.1869x; 1.1734x over previous
import functools

import jax
import jax.numpy as jnp
from jax.experimental import pallas as pl
from jax.experimental.pallas import tpu as pltpu

_CB = 4
_NBUF = 4


def _add_emb_kernel(x_hbm, table_ref, o_hbm, inb, outb, emb_ref, insem,
                    outsem, *, F, N, D, CB, NBUF):
    i = pl.program_id(0)
    nc = pl.num_programs(0)
    k = i % NBUF

    @pl.when(i == 0)
    def _prologue():
        for j in range(NBUF):
            pltpu.make_async_copy(
                x_hbm.at[pl.ds(j * CB, CB)], inb.at[j], insem.at[j]).start()
        t = table_ref[0:F, :]
        norm = jnp.sqrt(jnp.sum(t * t, axis=-1, keepdims=True))
        scale = jnp.where(norm > 1.0, 1.0 / jnp.maximum(norm, 1e-12), 1.0)
        t = t * scale
        r = jax.lax.broadcasted_iota(jnp.int32, (F * N, F), 0)
        c = jax.lax.broadcasted_iota(jnp.int32, (F * N, F), 1)
        sel = (c == (F - 1 - r // N)).astype(jnp.float32)
        emb_ref[...] = jnp.dot(sel, t, preferred_element_type=jnp.float32)

    pltpu.make_async_copy(
        x_hbm.at[pl.ds(i * CB, CB)], inb.at[k], insem.at[k]).wait()

    @pl.when(i >= NBUF)
    def _wait_out_slot():
        prev = i - NBUF
        pltpu.make_async_copy(
            outb.at[k], o_hbm.at[pl.ds(prev * CB, CB)], outsem.at[k]).wait()

    outb[k] = inb[k] + emb_ref[...][None]

    pltpu.make_async_copy(
        outb.at[k], o_hbm.at[pl.ds(i * CB, CB)], outsem.at[k]).start()

    nxt = i + NBUF

    @pl.when(nxt < nc)
    def _start_next_in():
        pltpu.make_async_copy(
            x_hbm.at[pl.ds(nxt * CB, CB)], inb.at[k], insem.at[k]).start()

    @pl.when(i == nc - 1)
    def _epilogue():
        for m in range(NBUF):
            cidx = nc - NBUF + m
            pltpu.make_async_copy(
                outb.at[m], o_hbm.at[pl.ds(cidx * CB, CB)],
                outsem.at[m]).wait()


def kernel(x, in_F, out_F, table):
    B, F, N, D = x.shape
    xv = x.reshape(B, F * N, D)
    nchunk = B // _CB
    out = pl.pallas_call(
        functools.partial(_add_emb_kernel, F=F, N=N, D=D, CB=_CB, NBUF=_NBUF),
        grid=(nchunk,),
        in_specs=[
            pl.BlockSpec(memory_space=pltpu.MemorySpace.HBM),
            pl.BlockSpec((table.shape[0], D), lambda i: (0, 0)),
        ],
        out_specs=pl.BlockSpec(memory_space=pltpu.MemorySpace.HBM),
        out_shape=jax.ShapeDtypeStruct((B, F * N, D), x.dtype),
        scratch_shapes=[
            pltpu.VMEM((_NBUF, _CB, F * N, D), jnp.float32),
            pltpu.VMEM((_NBUF, _CB, F * N, D), jnp.float32),
            pltpu.VMEM((F * N, D), jnp.float32),
            pltpu.SemaphoreType.DMA((_NBUF,)),
            pltpu.SemaphoreType.DMA((_NBUF,)),
        ],
    )(xv, table)
    return out.reshape(B, F, N, D)


# speedup vs baseline: 1.1902x; 1.0028x over previous
import functools

import jax
import jax.numpy as jnp
from jax.experimental import pallas as pl
from jax.experimental.pallas import tpu as pltpu

_CB = 2
_NBUF = 8


def _add_emb_kernel(x_hbm, table_ref, o_hbm, inb, outb, emb_ref, insem,
                    outsem, *, F, N, D, CB, NBUF):
    i = pl.program_id(0)
    nc = pl.num_programs(0)
    k = i % NBUF

    @pl.when(i == 0)
    def _prologue():
        for j in range(NBUF):
            pltpu.make_async_copy(
                x_hbm.at[pl.ds(j * CB, CB)], inb.at[j], insem.at[j]).start()
        t = table_ref[0:F, :]
        norm = jnp.sqrt(jnp.sum(t * t, axis=-1, keepdims=True))
        scale = jnp.where(norm > 1.0, 1.0 / jnp.maximum(norm, 1e-12), 1.0)
        t = t * scale
        r = jax.lax.broadcasted_iota(jnp.int32, (F * N, F), 0)
        c = jax.lax.broadcasted_iota(jnp.int32, (F * N, F), 1)
        sel = (c == (F - 1 - r // N)).astype(jnp.float32)
        emb_ref[...] = jnp.dot(sel, t, preferred_element_type=jnp.float32)

    pltpu.make_async_copy(
        x_hbm.at[pl.ds(i * CB, CB)], inb.at[k], insem.at[k]).wait()

    @pl.when(i >= NBUF)
    def _wait_out_slot():
        prev = i - NBUF
        pltpu.make_async_copy(
            outb.at[k], o_hbm.at[pl.ds(prev * CB, CB)], outsem.at[k]).wait()

    outb[k] = inb[k] + emb_ref[...][None]

    pltpu.make_async_copy(
        outb.at[k], o_hbm.at[pl.ds(i * CB, CB)], outsem.at[k]).start()

    nxt = i + NBUF

    @pl.when(nxt < nc)
    def _start_next_in():
        pltpu.make_async_copy(
            x_hbm.at[pl.ds(nxt * CB, CB)], inb.at[k], insem.at[k]).start()

    @pl.when(i == nc - 1)
    def _epilogue():
        for m in range(NBUF):
            cidx = nc - NBUF + m
            pltpu.make_async_copy(
                outb.at[m], o_hbm.at[pl.ds(cidx * CB, CB)],
                outsem.at[m]).wait()


def kernel(x, in_F, out_F, table):
    B, F, N, D = x.shape
    xv = x.reshape(B, F * N, D)
    nchunk = B // _CB
    out = pl.pallas_call(
        functools.partial(_add_emb_kernel, F=F, N=N, D=D, CB=_CB, NBUF=_NBUF),
        grid=(nchunk,),
        in_specs=[
            pl.BlockSpec(memory_space=pltpu.MemorySpace.HBM),
            pl.BlockSpec((table.shape[0], D), lambda i: (0, 0)),
        ],
        out_specs=pl.BlockSpec(memory_space=pltpu.MemorySpace.HBM),
        out_shape=jax.ShapeDtypeStruct((B, F * N, D), x.dtype),
        scratch_shapes=[
            pltpu.VMEM((_NBUF, _CB, F * N, D), jnp.float32),
            pltpu.VMEM((_NBUF, _CB, F * N, D), jnp.float32),
            pltpu.VMEM((F * N, D), jnp.float32),
            pltpu.SemaphoreType.DMA((_NBUF,)),
            pltpu.SemaphoreType.DMA((_NBUF,)),
        ],
    )(xv, table)
    return out.reshape(B, F, N, D)
